# baseline (speedup 1.0000x reference)
import functools

import jax
import jax.numpy as jnp
from jax import lax
from jax.experimental import pallas as pl
from jax.experimental.pallas import tpu as pltpu
from jax.experimental.pallas import tpu_sc as plsc

_LANES = 128
_OCT = 8


@functools.lru_cache(maxsize=None)
def _build(batch, hist, vocab, dim):
  info = plsc.get_sparse_core_info()
  nc, ns = info.num_cores, info.num_subcores
  nw = nc * ns
  assert batch % (nw * _OCT) == 0
  rows_per_w = batch // nw
  n_oct = rows_per_w // _OCT
  hist_p = (hist + 7) // 8 * 8

  mesh = plsc.VectorSubcoreMesh(core_axis_name="c", subcore_axis_name="s")

  @functools.partial(
      pl.kernel,
      out_type=jax.ShapeDtypeStruct((batch, hist, dim), jnp.float32),
      mesh=mesh,
      scratch_types=[
          pltpu.VMEM((rows_per_w, hist_p), jnp.int32),
          pltpu.VMEM((_OCT, hist_p, dim), jnp.float32),
      ]
      + [pltpu.SemaphoreType.DMA] * (2 * _OCT),
      compiler_params=pltpu.CompilerParams(use_tc_tiling_on_sc=False),
  )
  def k(idx_hbm, table_hbm, out_hbm, idx_v, rows_v, *sems):
    gsems, wsems = sems[:_OCT], sems[_OCT:]
    wid = lax.axis_index("s") * nc + lax.axis_index("c")
    base = wid * rows_per_w
    pltpu.sync_copy(
        idx_hbm.at[pl.ds(base, rows_per_w), pl.ds(0, hist_p)], idx_v
    )

    def body(q, carry):
      r0 = q * _OCT
      hs = []
      for i in range(_OCT):
        hs.append(
            pltpu.async_copy(
                table_hbm.at[idx_v.at[r0 + i]], rows_v.at[i], gsems[i]
            )
        )
      for i in range(_OCT):
        hs[i].wait()
        pltpu.sync_copy(
            rows_v.at[i, pl.ds(0, hist)], out_hbm.at[base + r0 + i]
        )
      return carry

    lax.fori_loop(0, n_oct, body, 0)

  return k


def kernel(input, table):
  batch, hist = input.shape
  vocab, dim = table.shape
  idx_pad = jnp.pad(input.astype(jnp.int32), ((0, 0), (0, _LANES - hist)))
  return _build(batch, hist, vocab, dim)(idx_pad, table)


# speedup vs baseline: 1.4498x; 1.4498x over previous
import functools

import jax
import jax.numpy as jnp
from jax import lax
from jax.experimental import pallas as pl
from jax.experimental.pallas import tpu as pltpu
from jax.experimental.pallas import tpu_sc as plsc

_LANES = 128
_OCT = 8


@functools.lru_cache(maxsize=None)
def _build(batch, hist, vocab, dim):
  info = plsc.get_sparse_core_info()
  nc, ns = info.num_cores, info.num_subcores
  nw = nc * ns
  assert batch % (nw * _OCT) == 0
  rows_per_w = batch // nw
  n_oct = rows_per_w // _OCT
  hist_p = (hist + 7) // 8 * 8

  mesh = plsc.VectorSubcoreMesh(core_axis_name="c", subcore_axis_name="s")

  @functools.partial(
      pl.kernel,
      out_type=jax.ShapeDtypeStruct((batch, hist, dim), jnp.float32),
      mesh=mesh,
      scratch_types=[
          pltpu.VMEM((rows_per_w, hist_p), jnp.int32),
          pltpu.VMEM((_OCT, hist_p, dim), jnp.float32),
      ]
      + [pltpu.SemaphoreType.DMA] * (2 * _OCT),
      compiler_params=pltpu.CompilerParams(use_tc_tiling_on_sc=False),
  )
  def k(idx_hbm, table_hbm, out_hbm, idx_v, rows_v, *sems):
    gsems, wsems = sems[:_OCT], sems[_OCT:]
    wid = lax.axis_index("s") * nc + lax.axis_index("c")
    base = wid * rows_per_w
    pltpu.sync_copy(
        idx_hbm.at[pl.ds(base, rows_per_w), pl.ds(0, hist_p)], idx_v
    )

    def body(q, carry):
      r0 = q * _OCT
      hs = []
      for i in range(_OCT):
        hs.append(
            pltpu.async_copy(
                table_hbm.at[idx_v.at[r0 + i]], rows_v.at[i], gsems[i]
            )
        )
      for i in range(_OCT):
        hs[i].wait()
        pltpu.sync_copy(
            rows_v.at[i, pl.ds(0, hist)], out_hbm.at[base + r0 + i]
        )
      return carry

    lax.fori_loop(0, n_oct, body, 0)

  return k


def kernel(input, table):
  batch, hist = input.shape
  vocab, dim = table.shape
  spread = (jnp.arange(_LANES - hist, dtype=jnp.int32) * 7919 + 63) % vocab
  idx_pad = jnp.concatenate(
      [input.astype(jnp.int32), jnp.broadcast_to(spread, (batch, _LANES - hist))],
      axis=1,
  )
  return _build(batch, hist, vocab, dim)(idx_pad, table)


# speedup vs baseline: 1.6750x; 1.1553x over previous
import functools

import jax
import jax.numpy as jnp
from jax import lax
from jax.experimental import pallas as pl
from jax.experimental.pallas import tpu as pltpu
from jax.experimental.pallas import tpu_sc as plsc

_LANES = 128
_OCT = 8


@functools.lru_cache(maxsize=None)
def _build(batch, hist, vocab, dim):
  info = plsc.get_sparse_core_info()
  nc, ns = info.num_cores, info.num_subcores
  nw = nc * ns
  assert batch % (nw * _OCT) == 0
  rows_per_w = batch // nw
  n_oct = rows_per_w // _OCT
  hist_p = (hist + 7) // 8 * 8

  mesh = plsc.VectorSubcoreMesh(core_axis_name="c", subcore_axis_name="s")

  @functools.partial(
      pl.kernel,
      out_type=jax.ShapeDtypeStruct((batch, hist, dim), jnp.float32),
      mesh=mesh,
      scratch_types=[
          pltpu.VMEM((rows_per_w, hist_p), jnp.int32),
          pltpu.VMEM((_OCT, hist_p, dim), jnp.float32),
      ]
      + [pltpu.SemaphoreType.DMA] * (2 * _OCT),
      compiler_params=pltpu.CompilerParams(use_tc_tiling_on_sc=False),
  )
  def k(idx_hbm, table_hbm, out_hbm, idx_v, rows_v, *sems):
    gsems, wsems = sems[:_OCT], sems[_OCT:]
    wid = lax.axis_index("s") * nc + lax.axis_index("c")
    base = wid * rows_per_w
    pltpu.sync_copy(
        idx_hbm.at[pl.ds(base, rows_per_w), pl.ds(0, hist_p)], idx_v
    )

    def body(q, carry):
      r0 = q * _OCT
      hs = []
      for i in range(_OCT):
        hs.append(
            pltpu.async_copy(
                table_hbm.at[idx_v.at[r0 + i]], rows_v.at[i], gsems[i]
            )
        )
      for i in range(_OCT):
        hs[i].wait()
        pltpu.sync_copy(
            rows_v.at[i, pl.ds(0, hist)], out_hbm.at[base + r0 + i]
        )
      return carry

    lax.fori_loop(0, n_oct, body, 0)

  return k


def kernel(input, table):
  batch, hist = input.shape
  vocab, dim = table.shape
  pad_w = _LANES - hist
  spread = (
      jax.lax.broadcasted_iota(jnp.int32, (batch, pad_w), 0) * 131
      + jax.lax.broadcasted_iota(jnp.int32, (batch, pad_w), 1) * 7919
  ) % vocab
  idx_pad = jnp.concatenate([input.astype(jnp.int32), spread], axis=1)
  return _build(batch, hist, vocab, dim)(idx_pad, table)


# speedup vs baseline: 1.8672x; 1.1147x over previous
import functools

import jax
import jax.numpy as jnp
from jax import lax
from jax.experimental import pallas as pl
from jax.experimental.pallas import tpu as pltpu
from jax.experimental.pallas import tpu_sc as plsc

_LANES = 128
_OCT = 8


@functools.lru_cache(maxsize=None)
def _build(batch, hist, vocab, dim):
  info = plsc.get_sparse_core_info()
  nc, ns = info.num_cores, info.num_subcores
  nw = nc * ns
  assert batch % (nw * _OCT) == 0
  rows_per_w = batch // nw
  n_oct = rows_per_w // _OCT
  hist_p = (hist + 7) // 8 * 8

  mesh = plsc.VectorSubcoreMesh(core_axis_name="c", subcore_axis_name="s")

  @functools.partial(
      pl.kernel,
      out_type=jax.ShapeDtypeStruct((batch, hist_p, 2 * dim), jnp.float32),
      mesh=mesh,
      scratch_types=[
          pltpu.VMEM((rows_per_w, hist_p), jnp.int32),
          pltpu.VMEM((_OCT, hist_p, dim), jnp.float32),
      ]
      + [pltpu.SemaphoreType.DMA] * (2 * _OCT),
      compiler_params=pltpu.CompilerParams(use_tc_tiling_on_sc=False),
  )
  def k(idx_hbm, table_hbm, out_hbm, idx_v, rows_v, *sems):
    gsems, wsems = sems[:_OCT], sems[_OCT:]
    wid = lax.axis_index("s") * nc + lax.axis_index("c")
    base = wid * rows_per_w
    pltpu.sync_copy(
        idx_hbm.at[pl.ds(base, rows_per_w), pl.ds(0, hist_p)], idx_v
    )

    def body(q, carry):
      r0 = q * _OCT
      hs = []
      for i in range(_OCT):
        hs.append(
            pltpu.async_copy(
                table_hbm.at[idx_v.at[r0 + i]], rows_v.at[i], gsems[i]
            )
        )
      for i in range(_OCT):
        hs[i].wait()
        pltpu.sync_copy(
            rows_v.at[i],
            out_hbm.at[base + r0 + i, pl.ds(0, hist_p), pl.ds(0, dim)],
        )
      return carry

    lax.fori_loop(0, n_oct, body, 0)

  return k


def kernel(input, table):
  batch, hist = input.shape
  vocab, dim = table.shape
  pad_w = _LANES - hist
  spread = (
      jax.lax.broadcasted_iota(jnp.int32, (batch, pad_w), 0) * 131
      + jax.lax.broadcasted_iota(jnp.int32, (batch, pad_w), 1) * 7919
  ) % vocab
  idx_pad = jnp.concatenate([input.astype(jnp.int32), spread], axis=1)
  out_p = _build(batch, hist, vocab, dim)(idx_pad, table)
  return out_p[:, :hist, :dim]


# speedup vs baseline: 1.8786x; 1.0061x over previous
import functools

import jax
import jax.numpy as jnp
from jax import lax
from jax.experimental import pallas as pl
from jax.experimental.pallas import tpu as pltpu
from jax.experimental.pallas import tpu_sc as plsc

_LANES = 128
_OCT = 16


@functools.lru_cache(maxsize=None)
def _build(batch, hist, vocab, dim):
  info = plsc.get_sparse_core_info()
  nc, ns = info.num_cores, info.num_subcores
  nw = nc * ns
  assert batch % (nw * _OCT) == 0
  rows_per_w = batch // nw
  n_oct = rows_per_w // _OCT
  hist_p = (hist + 7) // 8 * 8

  mesh = plsc.VectorSubcoreMesh(core_axis_name="c", subcore_axis_name="s")

  @functools.partial(
      pl.kernel,
      out_type=jax.ShapeDtypeStruct((batch, hist_p, 2 * dim), jnp.float32),
      mesh=mesh,
      scratch_types=[
          pltpu.VMEM((rows_per_w, hist_p), jnp.int32),
          pltpu.VMEM((_OCT, hist_p, dim), jnp.float32),
      ]
      + [pltpu.SemaphoreType.DMA] * (2 * _OCT),
      compiler_params=pltpu.CompilerParams(use_tc_tiling_on_sc=False),
  )
  def k(idx_hbm, table_hbm, out_hbm, idx_v, rows_v, *sems):
    gsems, wsems = sems[:_OCT], sems[_OCT:]
    wid = lax.axis_index("s") * nc + lax.axis_index("c")
    base = wid * rows_per_w
    pltpu.sync_copy(
        idx_hbm.at[pl.ds(base, rows_per_w), pl.ds(0, hist_p)], idx_v
    )

    def body(q, carry):
      r0 = q * _OCT
      hs = []
      for i in range(_OCT):
        hs.append(
            pltpu.async_copy(
                table_hbm.at[idx_v.at[r0 + i]], rows_v.at[i], gsems[i]
            )
        )
      for i in range(_OCT):
        hs[i].wait()
        pltpu.sync_copy(
            rows_v.at[i],
            out_hbm.at[base + r0 + i, pl.ds(0, hist_p), pl.ds(0, dim)],
        )
      return carry

    lax.fori_loop(0, n_oct, body, 0)

  return k


def kernel(input, table):
  batch, hist = input.shape
  vocab, dim = table.shape
  pad_w = _LANES - hist
  spread = (
      jax.lax.broadcasted_iota(jnp.int32, (batch, pad_w), 0) * 131
      + jax.lax.broadcasted_iota(jnp.int32, (batch, pad_w), 1) * 7919
  ) % vocab
  idx_pad = jnp.concatenate([input.astype(jnp.int32), spread], axis=1)
  out_p = _build(batch, hist, vocab, dim)(idx_pad, table)
  return out_p[:, :hist, :dim]
